# baseline (speedup 1.0000x reference)
import functools

import jax
import jax.numpy as jnp
from jax import lax
from jax.experimental import pallas as pl
from jax.experimental.pallas import tpu as pltpu
from jax.experimental.pallas import tpu_sc as plsc

D_MODEL = 768
D_FF = 3072
N_EXP = 8
TOP_K = 2
T = 2048
S = T * TOP_K
TILE_R = 256
G = S // TILE_R + N_EXP
LPAD = G * TILE_R
RB = S // TILE_R


def _router_body(x_ref, wg_ref, tok_ref, rw_ref, pos_ref, te_ref, tv_ref):
    f32 = jnp.float32
    x = x_ref[...]
    wg = wg_ref[...]

    logits8 = jax.lax.dot_general(wg, x, (((1,), (1,)), ((), ())),
                                  preferred_element_type=f32)
    m1 = jnp.max(logits8, axis=0, keepdims=True)
    a1 = jnp.argmax(logits8, axis=0, keepdims=True)
    row_e = jax.lax.broadcasted_iota(jnp.int32, logits8.shape, 0)
    masked = jnp.where(row_e == a1, -jnp.inf, logits8)
    m2 = jnp.max(masked, axis=0, keepdims=True)
    a2 = jnp.argmax(masked, axis=0, keepdims=True)
    e_row = jnp.concatenate([a1, a2], axis=1)

    logits_c = jax.lax.dot_general(x, wg, (((1,), (1,)), ((), ())),
                                   preferred_element_type=f32)
    m1c = jnp.max(logits_c, axis=1, keepdims=True)
    m2c = jnp.max(jnp.where(
        jax.lax.broadcasted_iota(jnp.int32, logits_c.shape, 1)
        == jnp.argmax(logits_c, axis=1, keepdims=True), -jnp.inf, logits_c),
        axis=1, keepdims=True)
    wf_c = 1.0 / (1.0 + jnp.exp(m2c - m1c))
    w_col = jnp.concatenate([wf_c, 1.0 - wf_c], axis=0)

    oh = jnp.where(e_row == jax.lax.broadcasted_iota(jnp.int32, (N_EXP, S), 0),
                   1.0, 0.0).astype(f32)
    su_i = jax.lax.broadcasted_iota(jnp.int32, (TILE_R, TILE_R), 0)
    su_j = jax.lax.broadcasted_iota(jnp.int32, (TILE_R, TILE_R), 1)
    su = jnp.where(su_i < su_j, 1.0, 0.0).astype(f32)
    runnings = []
    running = jnp.zeros((N_EXP, 1), f32)
    for b in range(RB):
        oh_b = oh[:, b * TILE_R:(b + 1) * TILE_R]
        runnings.append(running)
        running = running + jnp.sum(oh_b, axis=1, keepdims=True)
    counts = running
    pc = jnp.floor((counts + (TILE_R - 1)) * (1.0 / TILE_R)) * TILE_R
    sl_i = jax.lax.broadcasted_iota(jnp.int32, (N_EXP, N_EXP), 0)
    sl_j = jax.lax.broadcasted_iota(jnp.int32, (N_EXP, N_EXP), 1)
    sl = jnp.where(sl_j < sl_i, 1.0, 0.0).astype(f32)
    po = jax.lax.dot_general(sl, pc, (((1,), (0,)), ((), ())),
                             preferred_element_type=f32)

    dest_blocks = []
    for b in range(RB):
        oh_b = oh[:, b * TILE_R:(b + 1) * TILE_R]
        rank_b = jax.lax.dot_general(oh_b, su, (((1,), (0,)), ((), ())),
                                     preferred_element_type=f32)
        dest_b = jnp.sum(oh_b * (po + rank_b + runnings[b]),
                         axis=0, keepdims=True)
        dest_blocks.append(dest_b)
    dest = jnp.concatenate(dest_blocks, axis=1)
    pos_ref[...] = dest.astype(jnp.int32)

    ti = jax.lax.broadcasted_iota(jnp.int32, (S, 1), 0)
    tok_col = jnp.where(ti >= T, ti - T, ti).astype(f32)
    vals = jnp.concatenate([tok_col, w_col], axis=1)
    ridx = jax.lax.broadcasted_iota(jnp.int32, (TILE_R, S), 0)
    dest_i = dest.astype(jnp.int32)
    for g in range(G):
        a_g = jnp.where(dest_i == ridx + (g * TILE_R), 1.0, 0.0).astype(f32)
        out_g = jax.lax.dot_general(a_g, vals, (((1,), (0,)), ((), ())),
                                    precision=jax.lax.Precision.HIGHEST,
                                    preferred_element_type=f32)
        tok_ref[pl.ds(g * TILE_R, TILE_R), :] = out_g[:, 0:1].astype(jnp.int32)
        rw_ref[pl.ds(g * TILE_R, TILE_R), :] = out_g[:, 1:2]

    ends = po + pc
    gstart = (jax.lax.broadcasted_iota(jnp.int32, (1, G), 1)
              * TILE_R).astype(f32)
    te = jnp.sum(jnp.where(gstart >= ends, 1, 0), axis=0, keepdims=True)
    te_ref[...] = jnp.minimum(te, N_EXP - 1).astype(jnp.int32)
    total = jnp.sum(pc, axis=0, keepdims=True)
    tv_ref[...] = jnp.where(gstart < total, 1, 0).astype(jnp.int32)


def _router(x_flat, Wg):
    return pl.pallas_call(
        _router_body,
        out_shape=(jax.ShapeDtypeStruct((LPAD, 1), jnp.int32),
                   jax.ShapeDtypeStruct((LPAD, 1), jnp.float32),
                   jax.ShapeDtypeStruct((1, S), jnp.int32),
                   jax.ShapeDtypeStruct((1, G), jnp.int32),
                   jax.ShapeDtypeStruct((1, G), jnp.int32)),
    )(x_flat, Wg)


_NC, _NS = 2, 16
_NW = _NC * _NS
_BPW = LPAD // _NW
_CK = 16
_NCK = _BPW // _CK
_WU32 = D_MODEL // 2


@functools.cache
def _sc_gather_fn():
    @functools.partial(
        pl.kernel,
        out_type=jax.ShapeDtypeStruct((_NW, _NCK, _CK, _WU32), jnp.uint32),
        mesh=plsc.VectorSubcoreMesh(core_axis_name="c", subcore_axis_name="s",
                                    num_cores=_NC, num_subcores=_NS),
        scratch_types=[
            pltpu.VMEM((_NCK, _CK), jnp.int32),
            pltpu.VMEM((_NCK, _CK, _WU32), jnp.uint32),
            pltpu.SemaphoreType.DMA,
            pltpu.SemaphoreType.DMA,
        ],
    )
    def _gather(x_hbm, idx_hbm, out_hbm, idx_v, rows_v, gsem, osem):
        cid = lax.axis_index("c")
        sid = lax.axis_index("s")
        wid = sid * _NC + cid

        pltpu.sync_copy(idx_hbm.at[wid], idx_v)
        copies = [
            pltpu.async_copy(x_hbm.at[idx_v.at[k]], rows_v.at[k], gsem)
            for k in range(_NCK)
        ]
        for c in copies:
            c.wait()
        pltpu.sync_copy(rows_v, out_hbm.at[wid])

    return _gather


def _ffn_body(te_ref, tv_ref, xg_ref, w1_ref, b1_ref, w2_ref, b2_ref,
              rw_ref, out_ref):
    i = pl.program_id(0)

    @pl.when(tv_ref[i] == 1)
    def _():
        x = xg_ref[...]
        h = jax.lax.dot_general(x, w1_ref[0], (((1,), (1,)), ((), ())),
                                preferred_element_type=jnp.float32)
        h = h + b1_ref[0]
        h = 0.5 * h * (1.0 + jax.lax.erf(h * 0.7071067811865476))
        y = jax.lax.dot_general(h, w2_ref[0], (((1,), (1,)), ((), ())),
                                preferred_element_type=jnp.float32)
        y = y + b2_ref[0]
        out_ref[...] = y * rw_ref[...]

    @pl.when(tv_ref[i] == 0)
    def _():
        out_ref[...] = jnp.zeros_like(out_ref)


def _ffn(xg, W1, b1, W2, b2, rw, te, tv):
    grid_spec = pltpu.PrefetchScalarGridSpec(
        num_scalar_prefetch=2,
        grid=(G,),
        in_specs=[
            pl.BlockSpec((TILE_R, D_MODEL), lambda i, te, tv: (i, 0)),
            pl.BlockSpec((1, D_FF, D_MODEL), lambda i, te, tv: (te[i], 0, 0)),
            pl.BlockSpec((1, 1, D_FF), lambda i, te, tv: (te[i], 0, 0)),
            pl.BlockSpec((1, D_MODEL, D_FF), lambda i, te, tv: (te[i], 0, 0)),
            pl.BlockSpec((1, 1, D_MODEL), lambda i, te, tv: (te[i], 0, 0)),
            pl.BlockSpec((TILE_R, 1), lambda i, te, tv: (i, 0)),
        ],
        out_specs=pl.BlockSpec((TILE_R, D_MODEL), lambda i, te, tv: (i, 0)),
    )
    return pl.pallas_call(
        _ffn_body,
        grid_spec=grid_spec,
        out_shape=jax.ShapeDtypeStruct((LPAD, D_MODEL), jnp.float32),
    )(te, tv, xg, W1, b1.reshape(N_EXP, 1, D_FF), W2,
      b2.reshape(N_EXP, 1, D_MODEL), rw)


def kernel(x, Wg, W1, b1, W2, b2):
    B, Sq, D = x.shape
    x_flat = x.reshape(-1, D)
    row_token, row_w, pos, te, tv = _router(x_flat, Wg)
    xg = jnp.take(x_flat, row_token[:, 0], axis=0)
    y = _ffn(xg, W1, b1, W2, b2, row_w, te.reshape(G), tv.reshape(G))
    pos_flat = pos.reshape(S)
    out = (jnp.take(y, pos_flat[:T], axis=0)
           + jnp.take(y, pos_flat[T:], axis=0))
    return out.reshape(B, Sq, D)


# speedup vs baseline: 1.2832x; 1.2832x over previous
import functools

import jax
import jax.numpy as jnp
from jax import lax
from jax.experimental import pallas as pl
from jax.experimental.pallas import tpu as pltpu
from jax.experimental.pallas import tpu_sc as plsc

D_MODEL = 768
D_FF = 3072
N_EXP = 8
TOP_K = 2
T = 2048
S = T * TOP_K
TILE_R = 256
G = S // TILE_R + N_EXP
LPAD = G * TILE_R
RB = S // TILE_R


def _router_body(x_ref, wg_ref, tok_ref, rw_ref, pos_ref, te_ref, tv_ref):
    f32 = jnp.float32
    x = x_ref[...]
    wg = wg_ref[...]

    logits8 = jax.lax.dot_general(wg, x, (((1,), (1,)), ((), ())),
                                  preferred_element_type=f32)
    m1 = jnp.max(logits8, axis=0, keepdims=True)
    a1 = jnp.argmax(logits8, axis=0, keepdims=True)
    row_e = jax.lax.broadcasted_iota(jnp.int32, logits8.shape, 0)
    masked = jnp.where(row_e == a1, -jnp.inf, logits8)
    m2 = jnp.max(masked, axis=0, keepdims=True)
    a2 = jnp.argmax(masked, axis=0, keepdims=True)
    e_row = jnp.concatenate([a1, a2], axis=1)

    logits_c = jax.lax.dot_general(x, wg, (((1,), (1,)), ((), ())),
                                   preferred_element_type=f32)
    m1c = jnp.max(logits_c, axis=1, keepdims=True)
    m2c = jnp.max(jnp.where(
        jax.lax.broadcasted_iota(jnp.int32, logits_c.shape, 1)
        == jnp.argmax(logits_c, axis=1, keepdims=True), -jnp.inf, logits_c),
        axis=1, keepdims=True)
    wf_c = 1.0 / (1.0 + jnp.exp(m2c - m1c))
    w_col = jnp.concatenate([wf_c, 1.0 - wf_c], axis=0)

    oh = jnp.where(e_row == jax.lax.broadcasted_iota(jnp.int32, (N_EXP, S), 0),
                   1.0, 0.0).astype(f32)
    su_i = jax.lax.broadcasted_iota(jnp.int32, (TILE_R, TILE_R), 0)
    su_j = jax.lax.broadcasted_iota(jnp.int32, (TILE_R, TILE_R), 1)
    su = jnp.where(su_i < su_j, 1.0, 0.0).astype(f32)
    runnings = []
    running = jnp.zeros((N_EXP, 1), f32)
    for b in range(RB):
        oh_b = oh[:, b * TILE_R:(b + 1) * TILE_R]
        runnings.append(running)
        running = running + jnp.sum(oh_b, axis=1, keepdims=True)
    counts = running
    pc = jnp.floor((counts + (TILE_R - 1)) * (1.0 / TILE_R)) * TILE_R
    sl_i = jax.lax.broadcasted_iota(jnp.int32, (N_EXP, N_EXP), 0)
    sl_j = jax.lax.broadcasted_iota(jnp.int32, (N_EXP, N_EXP), 1)
    sl = jnp.where(sl_j < sl_i, 1.0, 0.0).astype(f32)
    po = jax.lax.dot_general(sl, pc, (((1,), (0,)), ((), ())),
                             preferred_element_type=f32)

    dest_blocks = []
    for b in range(RB):
        oh_b = oh[:, b * TILE_R:(b + 1) * TILE_R]
        rank_b = jax.lax.dot_general(oh_b, su, (((1,), (0,)), ((), ())),
                                     preferred_element_type=f32)
        dest_b = jnp.sum(oh_b * (po + rank_b + runnings[b]),
                         axis=0, keepdims=True)
        dest_blocks.append(dest_b)
    dest = jnp.concatenate(dest_blocks, axis=1)
    pos_ref[...] = dest.astype(jnp.int32)

    ti = jax.lax.broadcasted_iota(jnp.int32, (S, 1), 0)
    tok_i = jnp.where(ti >= T, ti - T, ti)
    tok_hi = (tok_i // 128).astype(f32)
    tok_lo = (tok_i % 128).astype(f32)
    w_hi = w_col.astype(jnp.bfloat16).astype(f32)
    w_lo = w_col - w_hi
    vals = jnp.concatenate([tok_hi, tok_lo, w_hi, w_lo], axis=1)
    ridx = jax.lax.broadcasted_iota(jnp.int32, (TILE_R, S), 0)
    dest_i = dest.astype(jnp.int32)
    for g in range(G):
        a_g = jnp.where(dest_i == ridx + (g * TILE_R), 1.0, 0.0).astype(f32)
        out_g = jax.lax.dot_general(a_g, vals, (((1,), (0,)), ((), ())),
                                    preferred_element_type=f32)
        tok_ref[pl.ds(g * TILE_R, TILE_R), :] = (
            out_g[:, 0:1] * 128.0 + out_g[:, 1:2]).astype(jnp.int32)
        rw_ref[pl.ds(g * TILE_R, TILE_R), :] = out_g[:, 2:3] + out_g[:, 3:4]

    ends = po + pc
    gstart = (jax.lax.broadcasted_iota(jnp.int32, (1, G), 1)
              * TILE_R).astype(f32)
    te = jnp.sum(jnp.where(gstart >= ends, 1, 0), axis=0, keepdims=True)
    te_ref[...] = jnp.minimum(te, N_EXP - 1).astype(jnp.int32)
    total = jnp.sum(pc, axis=0, keepdims=True)
    tv_ref[...] = jnp.where(gstart < total, 1, 0).astype(jnp.int32)


def _router(x_flat, Wg):
    return pl.pallas_call(
        _router_body,
        out_shape=(jax.ShapeDtypeStruct((LPAD, 1), jnp.int32),
                   jax.ShapeDtypeStruct((LPAD, 1), jnp.float32),
                   jax.ShapeDtypeStruct((1, S), jnp.int32),
                   jax.ShapeDtypeStruct((1, G), jnp.int32),
                   jax.ShapeDtypeStruct((1, G), jnp.int32)),
    )(x_flat, Wg)


_NC, _NS = 2, 16
_NW = _NC * _NS
_BPW = LPAD // _NW
_CK = 16
_NCK = _BPW // _CK
_WU32 = D_MODEL // 2


@functools.cache
def _sc_gather_fn():
    @functools.partial(
        pl.kernel,
        out_type=jax.ShapeDtypeStruct((_NW, _NCK, _CK, _WU32), jnp.uint32),
        mesh=plsc.VectorSubcoreMesh(core_axis_name="c", subcore_axis_name="s",
                                    num_cores=_NC, num_subcores=_NS),
        scratch_types=[
            pltpu.VMEM((_NCK, _CK), jnp.int32),
            pltpu.VMEM((_NCK, _CK, _WU32), jnp.uint32),
            pltpu.SemaphoreType.DMA,
            pltpu.SemaphoreType.DMA,
        ],
    )
    def _gather(x_hbm, idx_hbm, out_hbm, idx_v, rows_v, gsem, osem):
        cid = lax.axis_index("c")
        sid = lax.axis_index("s")
        wid = sid * _NC + cid

        pltpu.sync_copy(idx_hbm.at[wid], idx_v)
        copies = [
            pltpu.async_copy(x_hbm.at[idx_v.at[k]], rows_v.at[k], gsem)
            for k in range(_NCK)
        ]
        for c in copies:
            c.wait()
        pltpu.sync_copy(rows_v, out_hbm.at[wid])

    return _gather


def _ffn_body(te_ref, tv_ref, xg_ref, w1_ref, b1_ref, w2_ref, b2_ref,
              rw_ref, out_ref):
    i = pl.program_id(0)

    @pl.when(tv_ref[i] == 1)
    def _():
        x = xg_ref[...]
        h = jax.lax.dot_general(x, w1_ref[0], (((1,), (1,)), ((), ())),
                                preferred_element_type=jnp.float32)
        h = h + b1_ref[0]
        h = 0.5 * h * (1.0 + jax.lax.erf(h * 0.7071067811865476))
        y = jax.lax.dot_general(h, w2_ref[0], (((1,), (1,)), ((), ())),
                                preferred_element_type=jnp.float32)
        y = y + b2_ref[0]
        out_ref[...] = y * rw_ref[...]

    @pl.when(tv_ref[i] == 0)
    def _():
        out_ref[...] = jnp.zeros_like(out_ref)


def _ffn(xg, W1, b1, W2, b2, rw, te, tv):
    grid_spec = pltpu.PrefetchScalarGridSpec(
        num_scalar_prefetch=2,
        grid=(G,),
        in_specs=[
            pl.BlockSpec((TILE_R, D_MODEL), lambda i, te, tv: (i, 0)),
            pl.BlockSpec((1, D_FF, D_MODEL), lambda i, te, tv: (te[i], 0, 0)),
            pl.BlockSpec((1, 1, D_FF), lambda i, te, tv: (te[i], 0, 0)),
            pl.BlockSpec((1, D_MODEL, D_FF), lambda i, te, tv: (te[i], 0, 0)),
            pl.BlockSpec((1, 1, D_MODEL), lambda i, te, tv: (te[i], 0, 0)),
            pl.BlockSpec((TILE_R, 1), lambda i, te, tv: (i, 0)),
        ],
        out_specs=pl.BlockSpec((TILE_R, D_MODEL), lambda i, te, tv: (i, 0)),
    )
    return pl.pallas_call(
        _ffn_body,
        grid_spec=grid_spec,
        out_shape=jax.ShapeDtypeStruct((LPAD, D_MODEL), jnp.float32),
    )(te, tv, xg, W1, b1.reshape(N_EXP, 1, D_FF), W2,
      b2.reshape(N_EXP, 1, D_MODEL), rw)


def kernel(x, Wg, W1, b1, W2, b2):
    B, Sq, D = x.shape
    x_flat = x.reshape(-1, D)
    row_token, row_w, pos, te, tv = _router(x_flat, Wg)
    xg = jnp.take(x_flat, row_token[:, 0], axis=0)
    y = _ffn(xg, W1, b1, W2, b2, row_w, te.reshape(G), tv.reshape(G))
    pos_flat = pos.reshape(S)
    out = (jnp.take(y, pos_flat[:T], axis=0)
           + jnp.take(y, pos_flat[T:], axis=0))
    return out.reshape(B, Sq, D)


# speedup vs baseline: 1.3871x; 1.0810x over previous
import functools

import jax
import jax.numpy as jnp
from jax import lax
from jax.experimental import pallas as pl
from jax.experimental.pallas import tpu as pltpu
from jax.experimental.pallas import tpu_sc as plsc

D_MODEL = 768
D_FF = 3072
N_EXP = 8
TOP_K = 2
T = 2048
S = T * TOP_K
TILE_R = 256
G = S // TILE_R + N_EXP
LPAD = G * TILE_R
RB = S // TILE_R


def _router_body(x_ref, wg_ref, tok_ref, rw_ref, pos_ref, te_ref, tv_ref):
    f32 = jnp.float32
    x = x_ref[...]
    wg = wg_ref[...]

    logits8 = jax.lax.dot_general(wg, x, (((1,), (1,)), ((), ())),
                                  preferred_element_type=f32)
    m1 = jnp.max(logits8, axis=0, keepdims=True)
    a1 = jnp.argmax(logits8, axis=0, keepdims=True)
    row_e = jax.lax.broadcasted_iota(jnp.int32, logits8.shape, 0)
    masked = jnp.where(row_e == a1, -jnp.inf, logits8)
    m2 = jnp.max(masked, axis=0, keepdims=True)
    a2 = jnp.argmax(masked, axis=0, keepdims=True)
    e_row = jnp.concatenate([a1, a2], axis=1)

    logits_c = jax.lax.dot_general(x, wg, (((1,), (1,)), ((), ())),
                                   preferred_element_type=f32)
    m1c = jnp.max(logits_c, axis=1, keepdims=True)
    m2c = jnp.max(jnp.where(
        jax.lax.broadcasted_iota(jnp.int32, logits_c.shape, 1)
        == jnp.argmax(logits_c, axis=1, keepdims=True), -jnp.inf, logits_c),
        axis=1, keepdims=True)
    wf_c = 1.0 / (1.0 + jnp.exp(m2c - m1c))
    w_col = jnp.concatenate([wf_c, 1.0 - wf_c], axis=0)

    oh = jnp.where(e_row == jax.lax.broadcasted_iota(jnp.int32, (N_EXP, S), 0),
                   1.0, 0.0).astype(f32)
    su_i = jax.lax.broadcasted_iota(jnp.int32, (TILE_R, TILE_R), 0)
    su_j = jax.lax.broadcasted_iota(jnp.int32, (TILE_R, TILE_R), 1)
    su = jnp.where(su_i < su_j, 1.0, 0.0).astype(f32)
    runnings = []
    running = jnp.zeros((N_EXP, 1), f32)
    for b in range(RB):
        oh_b = oh[:, b * TILE_R:(b + 1) * TILE_R]
        runnings.append(running)
        running = running + jnp.sum(oh_b, axis=1, keepdims=True)
    counts = running
    pc = jnp.floor((counts + (TILE_R - 1)) * (1.0 / TILE_R)) * TILE_R
    sl_i = jax.lax.broadcasted_iota(jnp.int32, (N_EXP, N_EXP), 0)
    sl_j = jax.lax.broadcasted_iota(jnp.int32, (N_EXP, N_EXP), 1)
    sl = jnp.where(sl_j < sl_i, 1.0, 0.0).astype(f32)
    po = jax.lax.dot_general(sl, pc, (((1,), (0,)), ((), ())),
                             preferred_element_type=f32)

    dest_blocks = []
    for b in range(RB):
        oh_b = oh[:, b * TILE_R:(b + 1) * TILE_R]
        rank_b = jax.lax.dot_general(oh_b, su, (((1,), (0,)), ((), ())),
                                     preferred_element_type=f32)
        dest_b = jnp.sum(oh_b * (po + rank_b + runnings[b]),
                         axis=0, keepdims=True)
        dest_blocks.append(dest_b)
    dest = jnp.concatenate(dest_blocks, axis=1)
    pos_ref[...] = dest.astype(jnp.int32)

    ti = jax.lax.broadcasted_iota(jnp.int32, (S, 1), 0)
    tok_i = jnp.where(ti >= T, ti - T, ti)
    tok_hi = (tok_i // 128).astype(f32)
    tok_lo = (tok_i % 128).astype(f32)
    w_hi = w_col.astype(jnp.bfloat16).astype(f32)
    w_lo = w_col - w_hi
    vals = jnp.concatenate([tok_hi, tok_lo, w_hi, w_lo], axis=1)
    ridx = jax.lax.broadcasted_iota(jnp.int32, (TILE_R, S), 0)
    dest_i = dest.astype(jnp.int32)
    for g in range(G):
        a_g = jnp.where(dest_i == ridx + (g * TILE_R), 1.0, 0.0).astype(f32)
        out_g = jax.lax.dot_general(a_g, vals, (((1,), (0,)), ((), ())),
                                    preferred_element_type=f32)
        tok_ref[pl.ds(g * TILE_R, TILE_R), :] = (
            out_g[:, 0:1] * 128.0 + out_g[:, 1:2]).astype(jnp.int32)
        rw_ref[pl.ds(g * TILE_R, TILE_R), :] = out_g[:, 2:3] + out_g[:, 3:4]

    ends = po + pc
    gstart = (jax.lax.broadcasted_iota(jnp.int32, (1, G), 1)
              * TILE_R).astype(f32)
    te = jnp.sum(jnp.where(gstart >= ends, 1, 0), axis=0, keepdims=True)
    te_ref[...] = jnp.minimum(te, N_EXP - 1).astype(jnp.int32)
    total = jnp.sum(pc, axis=0, keepdims=True)
    tv_ref[...] = jnp.where(gstart < total, 1, 0).astype(jnp.int32)


def _router(x_flat, Wg):
    return pl.pallas_call(
        _router_body,
        out_shape=(jax.ShapeDtypeStruct((LPAD, 1), jnp.int32),
                   jax.ShapeDtypeStruct((LPAD, 1), jnp.float32),
                   jax.ShapeDtypeStruct((1, S), jnp.int32),
                   jax.ShapeDtypeStruct((1, G), jnp.int32),
                   jax.ShapeDtypeStruct((1, G), jnp.int32)),
    )(x_flat, Wg)


_NC, _NS = 2, 16
_NW = _NC * _NS
_BPW = LPAD // _NW
_CK = 16
_NCK = _BPW // _CK
_WU32 = D_MODEL // 2


@functools.cache
def _sc_gather_fn():
    @functools.partial(
        pl.kernel,
        out_type=jax.ShapeDtypeStruct((_NW, _NCK, _CK, _WU32), jnp.uint32),
        mesh=plsc.VectorSubcoreMesh(core_axis_name="c", subcore_axis_name="s",
                                    num_cores=_NC, num_subcores=_NS),
        scratch_types=[
            pltpu.VMEM((_NCK, _CK), jnp.int32),
            pltpu.VMEM((_NCK, _CK, _WU32), jnp.uint32),
            pltpu.SemaphoreType.DMA,
            pltpu.SemaphoreType.DMA,
        ],
    )
    def _gather(x_hbm, idx_hbm, out_hbm, idx_v, rows_v, gsem, osem):
        cid = lax.axis_index("c")
        sid = lax.axis_index("s")
        wid = sid * _NC + cid

        pltpu.sync_copy(idx_hbm.at[wid], idx_v)
        copies = [
            pltpu.async_copy(x_hbm.at[idx_v.at[k]], rows_v.at[k], gsem)
            for k in range(_NCK)
        ]
        for c in copies:
            c.wait()
        pltpu.sync_copy(rows_v, out_hbm.at[wid])

    return _gather


def _ffn_body(te_ref, tv_ref, xg_ref, w1_ref, b1_ref, w2_ref, b2_ref,
              rw_ref, out_ref):
    i = pl.program_id(0)

    @pl.when(tv_ref[i] == 1)
    def _():
        x = xg_ref[...]
        h = jax.lax.dot_general(x, w1_ref[0], (((1,), (1,)), ((), ())),
                                preferred_element_type=jnp.float32)
        h = h + b1_ref[0]
        h = 0.5 * h * (1.0 + jax.lax.erf(h * 0.7071067811865476))
        y = jax.lax.dot_general(h, w2_ref[0], (((1,), (1,)), ((), ())),
                                preferred_element_type=jnp.float32)
        y = y + b2_ref[0]
        out_ref[...] = y * rw_ref[...]

    @pl.when(tv_ref[i] == 0)
    def _():
        out_ref[...] = jnp.zeros_like(out_ref)


def _ffn(xg, W1, b1, W2, b2, rw, te, tv):
    grid_spec = pltpu.PrefetchScalarGridSpec(
        num_scalar_prefetch=2,
        grid=(G,),
        in_specs=[
            pl.BlockSpec((TILE_R, D_MODEL), lambda i, te, tv: (i, 0)),
            pl.BlockSpec((1, D_FF, D_MODEL), lambda i, te, tv: (te[i], 0, 0)),
            pl.BlockSpec((1, 1, D_FF), lambda i, te, tv: (te[i], 0, 0)),
            pl.BlockSpec((1, D_MODEL, D_FF), lambda i, te, tv: (te[i], 0, 0)),
            pl.BlockSpec((1, 1, D_MODEL), lambda i, te, tv: (te[i], 0, 0)),
            pl.BlockSpec((TILE_R, 1), lambda i, te, tv: (i, 0)),
        ],
        out_specs=pl.BlockSpec((TILE_R, D_MODEL), lambda i, te, tv: (i, 0)),
    )
    return pl.pallas_call(
        _ffn_body,
        grid_spec=grid_spec,
        out_shape=jax.ShapeDtypeStruct((LPAD, D_MODEL), jnp.float32),
    )(te, tv, xg, W1, b1.reshape(N_EXP, 1, D_FF), W2,
      b2.reshape(N_EXP, 1, D_MODEL), rw)


def kernel(x, Wg, W1, b1, W2, b2):
    B, Sq, D = x.shape
    x_flat = x.reshape(-1, D)
    row_token, row_w, pos, te, tv = _router(x_flat, Wg)
    xg = x_flat.at[row_token[:, 0]].get(mode="promise_in_bounds")
    y = _ffn(xg, W1, b1, W2, b2, row_w, te.reshape(G), tv.reshape(G))
    pos_flat = pos.reshape(S)
    out = (y.at[pos_flat[:T]].get(mode="promise_in_bounds")
           + y.at[pos_flat[T:]].get(mode="promise_in_bounds"))
    return out.reshape(B, Sq, D)
